# baseline (speedup 1.0000x reference)
import functools

import jax
import jax.numpy as jnp
from jax import lax
from jax.experimental import pallas as pl
from jax.experimental.pallas import tpu as pltpu
from jax.experimental.pallas import tpu_sc as plsc

NUM_CHARTS = 8
LATENT_DIM = 1024
RANK = 512
B = 2048
T = 256
NT = B // T
NI = NT + NUM_CHARTS - 1
NW = 32
BPW = B // NW


def _gather_rows(table, idx):
    D = table.shape[1]
    mesh = plsc.VectorSubcoreMesh(core_axis_name="c", subcore_axis_name="s")

    @functools.partial(
        pl.kernel,
        out_type=jax.ShapeDtypeStruct((B, D), table.dtype),
        mesh=mesh,
        scratch_types=[
            pltpu.VMEM((BPW,), jnp.int32),
            pltpu.VMEM((BPW, D), table.dtype),
            pltpu.SemaphoreType.DMA,
        ],
    )
    def gk(table_hbm, idx_hbm, out_hbm, idx_v, rows_v, sem):
        wid = lax.axis_index("s") * 2 + lax.axis_index("c")
        base = wid * BPW
        pltpu.sync_copy(idx_hbm.at[pl.ds(base, BPW)], idx_v)
        pltpu.async_copy(table_hbm.at[idx_v], rows_v, sem).wait()
        pltpu.sync_copy(rows_v, out_hbm.at[pl.ds(base, BPW)])

    return gk(table, idx)


def _routing(ids):
    perm = jnp.argsort(ids)
    inv = jnp.argsort(perm)
    counts = jnp.bincount(ids, length=NUM_CHARTS)
    off = jnp.concatenate([jnp.zeros((1,), jnp.int32),
                           jnp.cumsum(counts).astype(jnp.int32)])
    ft = off[:-1] // T
    lt = (off[1:] - 1) // T
    n_items = jnp.where(counts > 0, lt - ft + 1, 0)
    start = jnp.concatenate([jnp.zeros((1,), jnp.int32),
                             jnp.cumsum(n_items).astype(jnp.int32)])
    total = start[-1]
    g = jnp.arange(NI, dtype=jnp.int32)
    e = jnp.clip(jnp.searchsorted(start, g, side="right") - 1, 0, NUM_CHARTS - 1)
    e = e.astype(jnp.int32)
    tile = ft[e] + (g - start[e])
    valid = g < total
    tile = jnp.where(valid, tile, NT - 1).astype(jnp.int32)
    last_e = jnp.max(jnp.where(valid, e, -1)).astype(jnp.int32)
    e = jnp.where(valid, e, last_e)
    lo = jnp.clip(off[e] - tile * T, 0, T)
    hi = jnp.clip(off[e + 1] - tile * T, 0, T)
    lo = jnp.where(valid, lo, 0).astype(jnp.int32)
    hi = jnp.where(valid, hi, 0).astype(jnp.int32)
    first = jnp.concatenate([jnp.ones((1,), jnp.int32),
                             (tile[1:] != tile[:-1]).astype(jnp.int32)])
    meta = jnp.concatenate([tile, e, lo, hi, first]).astype(jnp.int32)
    return perm, inv, meta


def _gmm_body(n_out, meta_ref, x_ref, w_ref, bias_ref, out_ref):
    g = pl.program_id(0)
    lo = meta_ref[2 * NI + g]
    hi = meta_ref[3 * NI + g]
    first = meta_ref[4 * NI + g]

    @pl.when(lo < hi)
    def _():
        rowid = lax.broadcasted_iota(jnp.int32, (T, 1), 0)
        mask = (rowid >= lo) & (rowid < hi)
        xb = x_ref[...].astype(jnp.bfloat16)
        val = lax.dot_general(xb, w_ref[0], (((1,), (1,)), ((), ())),
                              preferred_element_type=jnp.float32)
        val = val + bias_ref[0]

        @pl.when(first == 1)
        def _():
            out_ref[...] = jnp.where(mask, val, 0.0)

        @pl.when(first == 0)
        def _():
            out_ref[...] = jnp.where(mask, val, out_ref[...])


def _grouped_matmul(meta, x, w, bias, n_in, n_out):
    return pl.pallas_call(
        functools.partial(_gmm_body, n_out),
        grid_spec=pltpu.PrefetchScalarGridSpec(
            num_scalar_prefetch=1,
            grid=(NI,),
            in_specs=[
                pl.BlockSpec((T, n_in), lambda g, m: (m[g], 0)),
                pl.BlockSpec((1, n_out, n_in), lambda g, m: (m[NI + g], 0, 0)),
                pl.BlockSpec((1, 1, n_out), lambda g, m: (m[NI + g], 0, 0)),
            ],
            out_specs=pl.BlockSpec((T, n_out), lambda g, m: (m[g], 0)),
        ),
        out_shape=jax.ShapeDtypeStruct((B, n_out), jnp.float32),
    )(meta, x, w, bias.reshape(NUM_CHARTS, 1, n_out))


@jax.jit
def kernel(z_n, source_idx, target_idx, W_enc, W_dec, c, d):
    wenc = W_enc.astype(jnp.bfloat16)
    wdec = W_dec.astype(jnp.bfloat16)
    src = source_idx.astype(jnp.int32)
    tgt = target_idx.astype(jnp.int32)

    perm_s, inv_s, meta_s = _routing(src)
    perm_t, inv_t, meta_t = _routing(tgt)
    idx_mid = inv_s[perm_t]

    z_s = _gather_rows(z_n, perm_s)
    h_s = _grouped_matmul(meta_s, z_s, wenc, c, LATENT_DIM, RANK)
    h_t = _gather_rows(h_s, idx_mid)
    y_t = _grouped_matmul(meta_t, h_t, wdec, d, RANK, LATENT_DIM)
    return _gather_rows(y_t, inv_t)


# speedup vs baseline: 1.0949x; 1.0949x over previous
import functools

import jax
import jax.numpy as jnp
from jax import lax
from jax.experimental import pallas as pl
from jax.experimental.pallas import tpu as pltpu
from jax.experimental.pallas import tpu_sc as plsc

NUM_CHARTS = 8
LATENT_DIM = 1024
RANK = 512
B = 2048
T = 256
NT = B // T
NI = NT + NUM_CHARTS - 1
NW = 32
BPW = B // NW


def _sc_permute(table, idx, scatter):
    D = table.shape[1]
    mesh = plsc.VectorSubcoreMesh(core_axis_name="c", subcore_axis_name="s")

    @functools.partial(
        pl.kernel,
        out_type=jax.ShapeDtypeStruct((B, D), table.dtype),
        mesh=mesh,
        scratch_types=[
            pltpu.VMEM((BPW,), jnp.int32),
            pltpu.VMEM((BPW, D), table.dtype),
            pltpu.SemaphoreType.DMA,
        ],
    )
    def gk(table_hbm, idx_hbm, out_hbm, idx_v, rows_v, sem):
        wid = lax.axis_index("s") * 2 + lax.axis_index("c")
        base = wid * BPW
        pltpu.sync_copy(idx_hbm.at[pl.ds(base, BPW)], idx_v)
        if scatter:
            pltpu.sync_copy(table_hbm.at[pl.ds(base, BPW)], rows_v)
            pltpu.async_copy(rows_v, out_hbm.at[idx_v], sem).wait()
        else:
            pltpu.async_copy(table_hbm.at[idx_v], rows_v, sem).wait()
            pltpu.sync_copy(rows_v, out_hbm.at[pl.ds(base, BPW)])

    return gk(table, idx)


def _routing(ids):
    eye = jnp.arange(NUM_CHARTS, dtype=jnp.int32)[None, :]
    oh = (ids[:, None] == eye).astype(jnp.int32)
    csum = jnp.cumsum(oh, axis=0)
    counts = csum[-1]
    off = jnp.concatenate([jnp.zeros((1,), jnp.int32),
                           jnp.cumsum(counts).astype(jnp.int32)])
    rank = jnp.sum((csum - 1) * oh, axis=1)
    base = jnp.sum(off[None, :NUM_CHARTS] * oh, axis=1)
    inv = (base + rank).astype(jnp.int32)

    ft = off[:-1] // T
    lt = (off[1:] - 1) // T
    n_items = jnp.where(counts > 0, lt - ft + 1, 0)
    start = jnp.concatenate([jnp.zeros((1,), jnp.int32),
                             jnp.cumsum(n_items).astype(jnp.int32)])
    total = start[-1]
    g = jnp.arange(NI, dtype=jnp.int32)
    e = jnp.clip(jnp.searchsorted(start, g, side="right") - 1, 0, NUM_CHARTS - 1)
    e = e.astype(jnp.int32)
    tile = ft[e] + (g - start[e])
    valid = g < total
    tile = jnp.where(valid, tile, NT - 1).astype(jnp.int32)
    last_e = jnp.max(jnp.where(valid, e, -1)).astype(jnp.int32)
    e = jnp.where(valid, e, last_e)
    lo = jnp.clip(off[e] - tile * T, 0, T)
    hi = jnp.clip(off[e + 1] - tile * T, 0, T)
    lo = jnp.where(valid, lo, 0).astype(jnp.int32)
    hi = jnp.where(valid, hi, 0).astype(jnp.int32)
    first = jnp.concatenate([jnp.ones((1,), jnp.int32),
                             (tile[1:] != tile[:-1]).astype(jnp.int32)])
    meta = jnp.concatenate([tile, e, lo, hi, first]).astype(jnp.int32)
    return inv, meta


def _gmm_body(out_dtype, meta_ref, x_ref, w_ref, bias_ref, out_ref):
    g = pl.program_id(0)
    lo = meta_ref[2 * NI + g]
    hi = meta_ref[3 * NI + g]
    first = meta_ref[4 * NI + g]

    @pl.when(lo < hi)
    def _():
        rowid = lax.broadcasted_iota(jnp.int32, (T, 1), 0)
        mask = (rowid >= lo) & (rowid < hi)
        xb = x_ref[...].astype(jnp.bfloat16)
        val = lax.dot_general(xb, w_ref[0], (((1,), (1,)), ((), ())),
                              preferred_element_type=jnp.float32)
        val = (val + bias_ref[0]).astype(out_dtype)

        @pl.when(first == 1)
        def _():
            out_ref[...] = jnp.where(mask, val, 0)

        @pl.when(first == 0)
        def _():
            out_ref[...] = jnp.where(mask, val, out_ref[...])


def _grouped_matmul(meta, x, w, bias, n_in, n_out, out_dtype):
    return pl.pallas_call(
        functools.partial(_gmm_body, out_dtype),
        grid_spec=pltpu.PrefetchScalarGridSpec(
            num_scalar_prefetch=1,
            grid=(NI,),
            in_specs=[
                pl.BlockSpec((T, n_in), lambda g, m: (m[g], 0)),
                pl.BlockSpec((1, n_out, n_in), lambda g, m: (m[NI + g], 0, 0)),
                pl.BlockSpec((1, 1, n_out), lambda g, m: (m[NI + g], 0, 0)),
            ],
            out_specs=pl.BlockSpec((T, n_out), lambda g, m: (m[g], 0)),
        ),
        out_shape=jax.ShapeDtypeStruct((B, n_out), out_dtype),
    )(meta, x, w, bias.reshape(NUM_CHARTS, 1, n_out))


@jax.jit
def kernel(z_n, source_idx, target_idx, W_enc, W_dec, c, d):
    wenc = W_enc.astype(jnp.bfloat16)
    wdec = W_dec.astype(jnp.bfloat16)
    src = source_idx.astype(jnp.int32)
    tgt = target_idx.astype(jnp.int32)

    inv_s, meta_s = _routing(src)
    inv_t, meta_t = _routing(tgt)
    mid = jnp.zeros((B,), jnp.int32).at[inv_s].set(inv_t)

    z_s = _sc_permute(z_n, inv_s, scatter=True)
    h_s = _grouped_matmul(meta_s, z_s, wenc, c, LATENT_DIM, RANK, jnp.float32)
    h_t = _sc_permute(h_s, mid, scatter=True)
    y_t = _grouped_matmul(meta_t, h_t, wdec, d, RANK, LATENT_DIM, jnp.float32)
    return _sc_permute(y_t, inv_t, scatter=False)


# speedup vs baseline: 1.1743x; 1.0725x over previous
import functools

import jax
import jax.numpy as jnp
from jax import lax
from jax.experimental import pallas as pl
from jax.experimental.pallas import tpu as pltpu
from jax.experimental.pallas import tpu_sc as plsc

NUM_CHARTS = 8
LATENT_DIM = 1024
RANK = 512
B = 2048
T = 256
NT = B // T
NI = NT + NUM_CHARTS - 1
NW = 32
BPW = B // NW


def _sc_permute(table, idx, scatter):
    D = table.shape[1]
    mesh = plsc.VectorSubcoreMesh(core_axis_name="c", subcore_axis_name="s")

    @functools.partial(
        pl.kernel,
        out_type=jax.ShapeDtypeStruct((B, D), table.dtype),
        mesh=mesh,
        scratch_types=[
            pltpu.VMEM((BPW,), jnp.int32),
            pltpu.VMEM((BPW, D), table.dtype),
            pltpu.SemaphoreType.DMA,
        ],
    )
    def gk(table_hbm, idx_hbm, out_hbm, idx_v, rows_v, sem):
        wid = lax.axis_index("s") * 2 + lax.axis_index("c")
        base = wid * BPW
        pltpu.sync_copy(idx_hbm.at[pl.ds(base, BPW)], idx_v)
        if scatter:
            pltpu.sync_copy(table_hbm.at[pl.ds(base, BPW)], rows_v)
            pltpu.async_copy(rows_v, out_hbm.at[idx_v], sem).wait()
        else:
            pltpu.async_copy(table_hbm.at[idx_v], rows_v, sem).wait()
            pltpu.sync_copy(rows_v, out_hbm.at[pl.ds(base, BPW)])

    return gk(table, idx)


def _sc_regroup(table, src_idx, dst_idx):
    D = table.shape[1]
    mesh = plsc.VectorSubcoreMesh(core_axis_name="c", subcore_axis_name="s")

    @functools.partial(
        pl.kernel,
        out_type=jax.ShapeDtypeStruct((B, D), table.dtype),
        mesh=mesh,
        scratch_types=[
            pltpu.VMEM((BPW,), jnp.int32),
            pltpu.VMEM((BPW,), jnp.int32),
            pltpu.VMEM((BPW, D), table.dtype),
            pltpu.SemaphoreType.DMA,
            pltpu.SemaphoreType.DMA,
        ],
    )
    def gk(table_hbm, sidx_hbm, didx_hbm, out_hbm, sidx_v, didx_v, rows_v, sem1, sem2):
        wid = lax.axis_index("s") * 2 + lax.axis_index("c")
        base = wid * BPW
        pltpu.sync_copy(sidx_hbm.at[pl.ds(base, BPW)], sidx_v)
        pltpu.sync_copy(didx_hbm.at[pl.ds(base, BPW)], didx_v)
        pltpu.async_copy(table_hbm.at[sidx_v], rows_v, sem1).wait()
        pltpu.async_copy(rows_v, out_hbm.at[didx_v], sem2).wait()

    return gk(table, src_idx, dst_idx)


def _routing(ids):
    eye = jnp.arange(NUM_CHARTS, dtype=jnp.int32)[None, :]
    oh = (ids[:, None] == eye).astype(jnp.int32)
    csum = jnp.cumsum(oh, axis=0)
    counts = csum[-1]
    off = jnp.concatenate([jnp.zeros((1,), jnp.int32),
                           jnp.cumsum(counts).astype(jnp.int32)])
    rank = jnp.sum((csum - 1) * oh, axis=1)
    base = jnp.sum(off[None, :NUM_CHARTS] * oh, axis=1)
    inv = (base + rank).astype(jnp.int32)

    ft = off[:-1] // T
    lt = (off[1:] - 1) // T
    n_items = jnp.where(counts > 0, lt - ft + 1, 0)
    start = jnp.concatenate([jnp.zeros((1,), jnp.int32),
                             jnp.cumsum(n_items).astype(jnp.int32)])
    total = start[-1]
    g = jnp.arange(NI, dtype=jnp.int32)
    e = jnp.clip(jnp.searchsorted(start, g, side="right") - 1, 0, NUM_CHARTS - 1)
    e = e.astype(jnp.int32)
    tile = ft[e] + (g - start[e])
    valid = g < total
    tile = jnp.where(valid, tile, NT - 1).astype(jnp.int32)
    last_e = jnp.max(jnp.where(valid, e, -1)).astype(jnp.int32)
    e = jnp.where(valid, e, last_e)
    lo = jnp.clip(off[e] - tile * T, 0, T)
    hi = jnp.clip(off[e + 1] - tile * T, 0, T)
    lo = jnp.where(valid, lo, 0).astype(jnp.int32)
    hi = jnp.where(valid, hi, 0).astype(jnp.int32)
    first = jnp.concatenate([jnp.ones((1,), jnp.int32),
                             (tile[1:] != tile[:-1]).astype(jnp.int32)])
    meta = jnp.concatenate([tile, e, lo, hi, first]).astype(jnp.int32)
    return inv, meta


def _gmm_body(out_dtype, meta_ref, x_ref, w_ref, bias_ref, out_ref):
    g = pl.program_id(0)
    lo = meta_ref[2 * NI + g]
    hi = meta_ref[3 * NI + g]
    first = meta_ref[4 * NI + g]

    @pl.when(lo < hi)
    def _():
        rowid = lax.broadcasted_iota(jnp.int32, (T, 1), 0)
        mask = (rowid >= lo) & (rowid < hi)
        xb = x_ref[...].astype(jnp.bfloat16)
        val = lax.dot_general(xb, w_ref[0], (((1,), (1,)), ((), ())),
                              preferred_element_type=jnp.float32)
        val = (val + bias_ref[0]).astype(out_dtype)

        @pl.when(first == 1)
        def _():
            out_ref[...] = jnp.where(mask, val, 0)

        @pl.when(first == 0)
        def _():
            out_ref[...] = jnp.where(mask, val, out_ref[...])


def _grouped_matmul(meta, x, w, bias, n_in, n_out, out_dtype):
    return pl.pallas_call(
        functools.partial(_gmm_body, out_dtype),
        grid_spec=pltpu.PrefetchScalarGridSpec(
            num_scalar_prefetch=1,
            grid=(NI,),
            in_specs=[
                pl.BlockSpec((T, n_in), lambda g, m: (m[g], 0)),
                pl.BlockSpec((1, n_out, n_in), lambda g, m: (m[NI + g], 0, 0)),
                pl.BlockSpec((1, 1, n_out), lambda g, m: (m[NI + g], 0, 0)),
            ],
            out_specs=pl.BlockSpec((T, n_out), lambda g, m: (m[g], 0)),
        ),
        out_shape=jax.ShapeDtypeStruct((B, n_out), out_dtype),
    )(meta, x, w, bias.reshape(NUM_CHARTS, 1, n_out))


@jax.jit
def kernel(z_n, source_idx, target_idx, W_enc, W_dec, c, d):
    wenc = W_enc.astype(jnp.bfloat16)
    wdec = W_dec.astype(jnp.bfloat16)
    src = source_idx.astype(jnp.int32)
    tgt = target_idx.astype(jnp.int32)

    inv_s, meta_s = _routing(src)
    inv_t, meta_t = _routing(tgt)

    z_s = _sc_permute(z_n, inv_s, scatter=True)
    h_s = _grouped_matmul(meta_s, z_s, wenc, c, LATENT_DIM, RANK, jnp.float32)
    h_t = _sc_regroup(h_s, inv_s, inv_t)
    y_t = _grouped_matmul(meta_t, h_t, wdec, d, RANK, LATENT_DIM, jnp.float32)
    return _sc_permute(y_t, inv_t, scatter=False)
